# baseline (speedup 1.0000x reference)
import functools

import jax
import jax.numpy as jnp
from jax import lax
from jax.experimental import pallas as pl
from jax.experimental.pallas import tpu as pltpu
from jax.experimental.pallas import tpu_sc as plsc

_B, _T, _D = 1024, 50, 300
_CHUNK = 2


def _sc_copy(v_hbm, o_hbm, buf0, buf1, sem_in, sem_out, nc):
    wid = lax.axis_index("s") * nc + lax.axis_index("c")
    rows_per_w = _B // (nc * 16)
    n = rows_per_w // _CHUNK
    base = wid * rows_per_w
    bufs = (buf0, buf1)

    ins = [
        pltpu.make_async_copy(
            v_hbm.at[pl.ds(base + i * _CHUNK, _CHUNK)], bufs[i % 2], sem_in.at[i % 2]
        )
        for i in range(n)
    ]
    outs = [
        pltpu.make_async_copy(
            bufs[i % 2], o_hbm.at[pl.ds(base + i * _CHUNK, _CHUNK)], sem_out.at[i % 2]
        )
        for i in range(n)
    ]

    ins[0].start()
    for i in range(n):
        ins[i].wait()
        outs[i].start()
        if i + 1 < n:
            if i >= 1:
                outs[i - 1].wait()
            ins[i + 1].start()
    if n >= 2:
        outs[n - 2].wait()
    outs[n - 1].wait()


def kernel(video, ques, attr, emb):
    del ques, attr, emb
    info = plsc.get_sparse_core_info()
    nc = info.num_cores
    mesh = plsc.VectorSubcoreMesh(core_axis_name="c", subcore_axis_name="s")
    k = pl.kernel(
        functools.partial(_sc_copy, nc=nc),
        out_type=jax.ShapeDtypeStruct((_B, _T, _D), jnp.float32),
        mesh=mesh,
        scratch_types=[
            pltpu.VMEM((_CHUNK, _T, _D), jnp.float32),
            pltpu.VMEM((_CHUNK, _T, _D), jnp.float32),
            pltpu.SemaphoreType.DMA((2,)),
            pltpu.SemaphoreType.DMA((2,)),
        ],
    )
    return k(video)
